# baseline (speedup 1.0000x reference)
import functools

import jax
import jax.numpy as jnp
from jax import lax
from jax.experimental import pallas as pl
from jax.experimental.pallas import tpu as pltpu
from jax.experimental.pallas import tpu_sc as plsc

BATCH = 16384
DIM = 32
LANES = 16
IDX_CHUNK = 128


def _make_sc_call():
    info = plsc.get_sparse_core_info()
    num_workers = info.num_cores * info.num_subcores
    b_per_w = BATCH // num_workers
    n_chunks = b_per_w // IDX_CHUNK
    mesh = plsc.VectorSubcoreMesh(core_axis_name="c", subcore_axis_name="s")

    @functools.partial(
        pl.kernel,
        mesh=mesh,
        compiler_params=pltpu.CompilerParams(
            needs_layout_passes=False, use_tc_tiling_on_sc=False),
        out_type=jax.ShapeDtypeStruct((BATCH,), jnp.float32),
        scratch_types=[
            pltpu.VMEM((n_chunks, IDX_CHUNK), jnp.int32),
            pltpu.VMEM((n_chunks, IDX_CHUNK), jnp.int32),
            pltpu.VMEM((b_per_w, DIM), jnp.float32),
            pltpu.VMEM((b_per_w, DIM), jnp.float32),
            pltpu.VMEM((DIM,), jnp.float32),
            pltpu.VMEM((LANES,), jnp.float32),
            pltpu.VMEM((b_per_w,), jnp.float32),
            pltpu.SemaphoreType.DMA,
        ],
    )
    def sc_call(uidx_hbm, midx_hbm, utab_hbm, mtab_hbm, w_hbm, b_hbm,
                out_hbm, uidx_v, midx_v, urows_v, mrows_v, w_v, b_v,
                out_v, sem):
        wid = lax.axis_index("s") * info.num_cores + lax.axis_index("c")
        base = wid * b_per_w
        row0 = wid * n_chunks

        pltpu.sync_copy(uidx_hbm.at[pl.ds(row0, n_chunks)], uidx_v)
        pltpu.sync_copy(midx_hbm.at[pl.ds(row0, n_chunks)], midx_v)
        pltpu.sync_copy(w_hbm, w_v)
        pltpu.sync_copy(b_hbm, b_v)

        copies = []
        for j in range(n_chunks):
            copies.append(pltpu.async_copy(
                utab_hbm.at[uidx_v.at[j]],
                urows_v.at[pl.ds(j * IDX_CHUNK, IDX_CHUNK)], sem))
            copies.append(pltpu.async_copy(
                mtab_hbm.at[midx_v.at[j]],
                mrows_v.at[pl.ds(j * IDX_CHUNK, IDX_CHUNK)], sem))
        for c in copies:
            c.wait()

        w_lo = w_v[pl.ds(0, LANES)]
        w_hi = w_v[pl.ds(LANES, LANES)]
        bias = b_v[pl.ds(0, LANES)]
        lane = lax.iota(jnp.int32, LANES)

        def body(c, _):
            rows = lane + c * LANES
            acc = bias
            for d in range(DIM):
                col = jnp.full((LANES,), d, dtype=jnp.int32)
                ug = plsc.load_gather(urows_v, [rows, col])
                mg = plsc.load_gather(mrows_v, [rows, col])
                wd = w_lo[d] if d < LANES else w_hi[d - LANES]
                acc = acc + ug * mg * wd
            out_v[pl.ds(c * LANES, LANES)] = acc
            return 0

        lax.fori_loop(0, b_per_w // LANES, body, 0)

        pltpu.sync_copy(out_v, out_hbm.at[pl.ds(base, b_per_w)])

    return sc_call


_SC_CALL = None


def kernel(user_indices, movie_indices, user_table, movie_table, W, b):
    global _SC_CALL
    if _SC_CALL is None:
        _SC_CALL = _make_sc_call()
    uidx = user_indices.astype(jnp.int32).reshape(BATCH // IDX_CHUNK, IDX_CHUNK)
    midx = movie_indices.astype(jnp.int32).reshape(BATCH // IDX_CHUNK, IDX_CHUNK)
    w_flat = W.reshape(DIM)
    b_vec = jnp.broadcast_to(b.reshape(()), (LANES,))
    out = _SC_CALL(uidx, midx, user_table, movie_table, w_flat, b_vec)
    return out.reshape(BATCH, 1)


# speedup vs baseline: 3.3115x; 3.3115x over previous
import functools

import jax
import jax.numpy as jnp
from jax import lax
from jax.experimental import pallas as pl
from jax.experimental.pallas import tpu as pltpu
from jax.experimental.pallas import tpu_sc as plsc

BATCH = 16384
DIM = 32
LANES = 16
NUM_USERS = 100000
NUM_MOVIES = 1000000


def _make_main_call():
    info = plsc.get_sparse_core_info()
    nc, ns = info.num_cores, info.num_subcores
    b_per_s = BATCH // ns
    blocks_per_core = DIM // 8 // nc
    mesh = plsc.VectorSubcoreMesh(core_axis_name="c", subcore_axis_name="s")

    @functools.partial(
        pl.kernel,
        mesh=mesh,
        compiler_params=pltpu.CompilerParams(needs_layout_passes=False),
        out_type=jax.ShapeDtypeStruct((nc, BATCH), jnp.float32),
        scratch_types=[
            pltpu.VMEM_SHARED((NUM_MOVIES,), jnp.float32),
            pltpu.VMEM_SHARED((NUM_USERS,), jnp.float32),
            pltpu.VMEM((b_per_s,), jnp.int32),
            pltpu.VMEM((b_per_s,), jnp.int32),
            pltpu.VMEM((b_per_s,), jnp.float32),
            pltpu.VMEM((b_per_s,), jnp.float32),
            pltpu.VMEM((b_per_s,), jnp.float32),
            pltpu.VMEM((DIM,), jnp.float32),
            pltpu.SemaphoreType.DMA,
        ],
    )
    def main_call(uidx_hbm, midx_hbm, utab_hbm, mtab_hbm, w_hbm, out_hbm,
                  spm_m, spm_u, uidx_v, midx_v, gu_v, gm_v, acc_v, w_v, sem):
        c = lax.axis_index("c")
        s = lax.axis_index("s")
        base = s * b_per_s

        pltpu.sync_copy(uidx_hbm.at[pl.ds(base, b_per_s)], uidx_v)
        pltpu.sync_copy(midx_hbm.at[pl.ds(base, b_per_s)], midx_v)
        pltpu.sync_copy(w_hbm, w_v)

        for k in range(b_per_s // LANES):
            sl = pl.ds(k * LANES, LANES)
            acc_v[sl] = jnp.zeros((LANES,), jnp.float32)

        w_lo = w_v[pl.ds(0, LANES)]
        w_hi = w_v[pl.ds(LANES, LANES)]

        for a_local in range(blocks_per_core):
            for f in range(8):
                d_local = a_local * 8 + f
                @pl.when(s == 0)
                def _stage():
                    pltpu.sync_copy(
                        mtab_hbm.at[c * blocks_per_core + a_local, f], spm_m)
                    pltpu.sync_copy(
                        utab_hbm.at[c * blocks_per_core + a_local, f], spm_u)
                plsc.subcore_barrier()

                pltpu.async_copy(spm_u.at[uidx_v], gu_v, sem).wait()
                pltpu.async_copy(spm_m.at[midx_v], gm_v, sem).wait()

                wd = lax.select(c == 0, w_lo, w_hi)[d_local]
                for k in range(b_per_s // LANES):
                    sl = pl.ds(k * LANES, LANES)
                    acc_v[sl] = acc_v[sl] + gu_v[sl] * gm_v[sl] * wd
                plsc.subcore_barrier()

        pltpu.sync_copy(acc_v, out_hbm.at[c, pl.ds(base, b_per_s)])

    return main_call


def _make_combine_call():
    info = plsc.get_sparse_core_info()
    num_workers = info.num_cores * info.num_subcores
    b_per_w = BATCH // num_workers
    mesh = plsc.VectorSubcoreMesh(core_axis_name="c", subcore_axis_name="s")

    @functools.partial(
        pl.kernel,
        mesh=mesh,
        compiler_params=pltpu.CompilerParams(needs_layout_passes=False),
        out_type=jax.ShapeDtypeStruct((BATCH,), jnp.float32),
        scratch_types=[
            pltpu.VMEM((b_per_w,), jnp.float32),
            pltpu.VMEM((b_per_w,), jnp.float32),
            pltpu.VMEM((b_per_w,), jnp.float32),
            pltpu.VMEM((LANES,), jnp.float32),
        ],
    )
    def combine_call(part_hbm, b_hbm, out_hbm, p0_v, p1_v, o_v, b_v):
        wid = lax.axis_index("s") * info.num_cores + lax.axis_index("c")
        base = wid * b_per_w
        pltpu.sync_copy(part_hbm.at[0, pl.ds(base, b_per_w)], p0_v)
        pltpu.sync_copy(part_hbm.at[1, pl.ds(base, b_per_w)], p1_v)
        pltpu.sync_copy(b_hbm, b_v)
        bias = b_v[pl.ds(0, LANES)]
        for k in range(b_per_w // LANES):
            sl = pl.ds(k * LANES, LANES)
            o_v[sl] = p0_v[sl] + p1_v[sl] + bias
        pltpu.sync_copy(o_v, out_hbm.at[pl.ds(base, b_per_w)])

    return combine_call


_MAIN_CALL = None
_COMBINE_CALL = None


def kernel(user_indices, movie_indices, user_table, movie_table, W, b):
    global _MAIN_CALL, _COMBINE_CALL
    if _MAIN_CALL is None:
        _MAIN_CALL = _make_main_call()
        _COMBINE_CALL = _make_combine_call()
    uidx = user_indices.astype(jnp.int32)
    midx = movie_indices.astype(jnp.int32)
    ut3 = user_table.T.reshape(4, 8, NUM_USERS)
    mt3 = movie_table.T.reshape(4, 8, NUM_MOVIES)
    w_flat = W.reshape(DIM)
    b_vec = jnp.broadcast_to(b.reshape(()), (LANES,))
    parts = _MAIN_CALL(uidx, midx, ut3, mt3, w_flat)
    out = _COMBINE_CALL(parts, b_vec)
    return out.reshape(BATCH, 1)
